# Initial kernel scaffold; baseline (speedup 1.0000x reference)
import jax
import jax.numpy as jnp
from jax.experimental import pallas as pl


def kernel(users, tables):
    raise NotImplementedError("write your pallas kernel here")


# speedup vs baseline: 1.2085x; 1.2085x over previous
import functools

import jax
import jax.numpy as jnp
from jax import lax
from jax.experimental import pallas as pl
from jax.experimental.pallas import tpu as pltpu
from jax.experimental.pallas import tpu_sc as plsc

_NC = 2
_NS = 16
_NW = _NC * _NS


def _gather_call(table2d, flat_idx, rows, dim):
    rows_per_w = rows // _NW
    chunk = 1024
    n_chunks = rows_per_w // chunk

    mesh = plsc.VectorSubcoreMesh(
        core_axis_name="c", subcore_axis_name="s",
        num_cores=_NC, num_subcores=_NS)

    @functools.partial(
        pl.kernel,
        mesh=mesh,
        out_type=jax.ShapeDtypeStruct((rows, dim), jnp.float32),
        scratch_types=[
            pltpu.VMEM((rows_per_w,), jnp.int32),
            pltpu.VMEM((chunk, dim), jnp.float32),
            pltpu.SemaphoreType.DMA,
        ],
        compiler_params=pltpu.CompilerParams(use_tc_tiling_on_sc=False),
    )
    def gather_k(table_hbm, idx_hbm, out_hbm, idx_v, rows_v, sem):
        wid = lax.axis_index("s") * _NC + lax.axis_index("c")
        base = pl.multiple_of(wid * rows_per_w, 8)
        pltpu.sync_copy(idx_hbm.at[pl.ds(base, rows_per_w)], idx_v)
        for c in range(n_chunks):
            pltpu.async_copy(
                table_hbm.at[idx_v.at[pl.ds(c * chunk, chunk)]],
                rows_v, sem).wait()
            pltpu.sync_copy(rows_v, out_hbm.at[pl.ds(base + c * chunk, chunk)])

    return gather_k(table2d, flat_idx)


def kernel(users, tables):
    num_fields, vocab, dim = tables.shape
    batch = users.shape[0]
    rows = batch * num_fields

    offsets = jnp.arange(num_fields, dtype=jnp.int32) * vocab
    flat_idx = (users.astype(jnp.int32) + offsets[None, :]).reshape(rows)
    table2d = tables.reshape(num_fields * vocab, dim)

    out = _gather_call(table2d, flat_idx, rows, dim)
    return out.reshape(batch, num_fields * dim)


# speedup vs baseline: 1.2165x; 1.0067x over previous
import functools

import jax
import jax.numpy as jnp
from jax import lax
from jax.experimental import pallas as pl
from jax.experimental.pallas import tpu as pltpu
from jax.experimental.pallas import tpu_sc as plsc

_NC = 2
_NS = 16
_NW = _NC * _NS


def _gather_call(table2d, flat_idx, rows, dim):
    rows_per_w = rows // _NW
    chunk = 1024
    n_chunks = rows_per_w // chunk
    nbuf = 3

    mesh = plsc.VectorSubcoreMesh(
        core_axis_name="c", subcore_axis_name="s",
        num_cores=_NC, num_subcores=_NS)

    @functools.partial(
        pl.kernel,
        mesh=mesh,
        out_type=jax.ShapeDtypeStruct((rows, dim), jnp.float32),
        scratch_types=[
            pltpu.VMEM((rows_per_w,), jnp.int32),
            pltpu.VMEM((nbuf, chunk, dim), jnp.float32),
            pltpu.SemaphoreType.DMA((nbuf,)),
            pltpu.SemaphoreType.DMA((nbuf,)),
        ],
        compiler_params=pltpu.CompilerParams(use_tc_tiling_on_sc=False),
    )
    def gather_k(table_hbm, idx_hbm, out_hbm, idx_v, bufs, gsem, osem):
        wid = lax.axis_index("s") * _NC + lax.axis_index("c")
        base = pl.multiple_of(wid * rows_per_w, 8)
        pltpu.sync_copy(idx_hbm.at[pl.ds(base, rows_per_w)], idx_v)

        def start_gather(c, j):
            pltpu.async_copy(
                table_hbm.at[idx_v.at[pl.ds(c * chunk, chunk)]],
                bufs.at[j], gsem.at[j])

        def start_write(c, j):
            pltpu.async_copy(
                bufs.at[j], out_hbm.at[pl.ds(base + c * chunk, chunk)],
                osem.at[j])

        for c in range(nbuf):
            start_gather(c, c)
        for c in range(n_chunks):
            j = c % nbuf
            pltpu.make_async_copy(
                table_hbm.at[idx_v.at[pl.ds(c * chunk, chunk)]],
                bufs.at[j], gsem.at[j]).wait()
            start_write(c, j)
            nxt = c + nbuf
            if nxt < n_chunks:
                pltpu.make_async_copy(
                    bufs.at[j], out_hbm.at[pl.ds(base + c * chunk, chunk)],
                    osem.at[j]).wait()
                start_gather(nxt, j)
            else:
                pltpu.make_async_copy(
                    bufs.at[j], out_hbm.at[pl.ds(base + c * chunk, chunk)],
                    osem.at[j]).wait()

    return gather_k(table2d, flat_idx)


def kernel(users, tables):
    num_fields, vocab, dim = tables.shape
    batch = users.shape[0]
    rows = batch * num_fields

    offsets = jnp.arange(num_fields, dtype=jnp.int32) * vocab
    flat_idx = (users.astype(jnp.int32) + offsets[None, :]).reshape(rows)
    table2d = tables.reshape(num_fields * vocab, dim)

    out = _gather_call(table2d, flat_idx, rows, dim)
    return out.reshape(batch, num_fields * dim)


# speedup vs baseline: 3.8504x; 3.1650x over previous
import functools

import jax
import jax.numpy as jnp
from jax import lax
from jax.experimental import pallas as pl
from jax.experimental.pallas import tpu as pltpu
from jax.experimental.pallas import tpu_sc as plsc

_NC = 2
_NS = 16
_NW = _NC * _NS


def _lookup_call(tables_t, users_t, num_fields, vocab, dim, batch):
    mesh = plsc.VectorSubcoreMesh(
        core_axis_name="c", subcore_axis_name="s",
        num_cores=_NC, num_subcores=_NS)

    @functools.partial(
        pl.kernel,
        mesh=mesh,
        out_type=jax.ShapeDtypeStruct((num_fields * dim, batch), jnp.float32),
        scratch_types=[
            pltpu.VMEM((vocab,), jnp.float32),
            pltpu.VMEM((batch // 2,), jnp.int32),
            pltpu.VMEM((batch,), jnp.float32),
            pltpu.SemaphoreType.DMA,
        ],
        compiler_params=pltpu.CompilerParams(needs_layout_passes=False),
    )
    def lookup_k(t_hbm, u_hbm, out_hbm, drow_v, idx_v, orow_v, sem):
        wid = lax.axis_index("s") * _NC + lax.axis_index("c")
        half = batch // 2
        for f in range(num_fields):
            pltpu.sync_copy(t_hbm.at[f, wid], drow_v)
            for h in range(2):
                pltpu.sync_copy(u_hbm.at[f, pl.ds(h * half, half)], idx_v)

                def body(j, _, h=h):
                    u = idx_v[pl.ds(j * 16, 16)]
                    orow_v[pl.ds(h * half + j * 16, 16)] = (
                        plsc.load_gather(drow_v, [u]))
                    return 0

                lax.fori_loop(0, half // 16, body, 0)
            pltpu.sync_copy(orow_v, out_hbm.at[f * dim + wid])

    return lookup_k(tables_t, users_t)


def kernel(users, tables):
    num_fields, vocab, dim = tables.shape
    batch = users.shape[0]

    tables_t = jnp.transpose(tables, (0, 2, 1))
    users_t = jnp.transpose(users.astype(jnp.int32), (1, 0))

    out_t = _lookup_call(tables_t, users_t, num_fields, vocab, dim, batch)
    return jnp.transpose(out_t, (1, 0)).reshape(batch, num_fields * dim)


# speedup vs baseline: 7.2361x; 1.8793x over previous
import functools

import jax
import jax.numpy as jnp
from jax import lax
from jax.experimental import pallas as pl
from jax.experimental.pallas import tpu as pltpu
from jax.experimental.pallas import tpu_sc as plsc

_NC = 2
_NS = 16
_NW = _NC * _NS


def _lookup_call(tables_t, users_t, num_fields, vocab, dim, batch):
    mesh = plsc.VectorSubcoreMesh(
        core_axis_name="c", subcore_axis_name="s",
        num_cores=_NC, num_subcores=_NS)

    @functools.partial(
        pl.kernel,
        mesh=mesh,
        out_type=jax.ShapeDtypeStruct((num_fields * dim, batch), jnp.float32),
        scratch_types=[
            pltpu.VMEM((vocab,), jnp.float32),
            pltpu.VMEM((batch // 2,), jnp.int32),
            pltpu.VMEM((batch,), jnp.float32),
            pltpu.SemaphoreType.DMA((2,)),
        ],
        compiler_params=pltpu.CompilerParams(needs_layout_passes=False),
    )
    def lookup_k(t_hbm, u_hbm, out_hbm, drow_v, idx_v, orow_v, osem):
        wid = lax.axis_index("s") * _NC + lax.axis_index("c")
        half = batch // 2

        def owrite(f, h):
            return pltpu.make_async_copy(
                orow_v.at[pl.ds(h * half, half)],
                out_hbm.at[f * dim + wid, pl.ds(h * half, half)],
                osem.at[h])

        for f in range(num_fields):
            pltpu.sync_copy(t_hbm.at[f, wid], drow_v)
            for h in range(2):
                pltpu.sync_copy(u_hbm.at[f, pl.ds(h * half, half)], idx_v)
                if f > 0:
                    owrite(f - 1, h).wait()

                @functools.partial(
                    plsc.parallel_loop, 0, half // 16, unroll=8)
                def body(j, h=h):
                    u = idx_v[pl.ds(j * 16, 16)]
                    orow_v[pl.ds(h * half + j * 16, 16)] = (
                        plsc.load_gather(drow_v, [u]))

                owrite(f, h).start()
        for h in range(2):
            owrite(num_fields - 1, h).wait()

    return lookup_k(tables_t, users_t)


def kernel(users, tables):
    num_fields, vocab, dim = tables.shape
    batch = users.shape[0]

    tables_t = jnp.transpose(tables, (0, 2, 1))
    users_t = jnp.transpose(users.astype(jnp.int32), (1, 0))

    out_t = _lookup_call(tables_t, users_t, num_fields, vocab, dim, batch)
    return jnp.transpose(out_t, (1, 0)).reshape(batch, num_fields * dim)
